# baseline (speedup 1.0000x reference)
import jax
import jax.numpy as jnp
from jax import lax
from jax.experimental import pallas as pl
from jax.experimental.pallas import tpu as pltpu
from jax.experimental.pallas import tpu_sc as plsc

N = 10000
E = 320000
DF = 128
H = 16
NPAD = 10240
NC = 2
NS = 16
NW = NC * NS
EW = E // NW
K = 400
SUB = 80
NSUB = K // SUB
NCHUNK = EW // K
ZR = NPAD // NS

_mesh = plsc.VectorSubcoreMesh(
    core_axis_name="c", subcore_axis_name="s", num_cores=NC, num_subcores=NS)
_sc_params = pltpu.CompilerParams(
    needs_layout_passes=False, use_tc_tiling_on_sc=False)


def _zero16():
    return jnp.zeros((16,), jnp.float32)


_NBUF = 3


def _sc_layer_body(xl_hbm, xr_hbm, e_hbm, src_hbm, dst_hbm, att_hbm,
                   denomp_hbm, outp_hbm,
                   srcall, dstall,
                   xlr0, xrr0, er0, expb0,
                   xlr1, xrr1, er1, expb1,
                   xlr2, xrr2, er2, expb2,
                   attv, zbuf, zrowb, shared_den, shared_out,
                   seml0, seml1, seml2, sems0, sems1, sems2):
    c = lax.axis_index("c")
    s = lax.axis_index("s")
    wid = c * NS + s
    base0 = wid * EW

    bufs = ((xlr0, xrr0, er0, expb0, seml0, sems0),
            (xlr1, xrr1, er1, expb1, seml1, sems1),
            (xlr2, xrr2, er2, expb2, seml2, sems2))

    pltpu.sync_copy(src_hbm.at[pl.ds(base0, EW)], srcall)
    pltpu.sync_copy(dst_hbm.at[pl.ds(base0 // SUB, EW // SUB)], dstall)

    z = _zero16()

    def _zb(i, _):
        zbuf[pl.ds(i * 16, 16)] = z
        return 0
    lax.fori_loop(0, ZR // 16, _zb, 0)

    def _zr(i, _):
        zrowb[i, :] = z
        return 0
    lax.fori_loop(0, ZR, _zr, 0)
    pltpu.sync_copy(zbuf, shared_den.at[pl.ds(s * ZR, ZR)])
    pltpu.sync_copy(zrowb, shared_out.at[pl.ds(s * ZR, ZR)])

    pltpu.sync_copy(att_hbm, attv)
    att_vec = attv[...]
    att_spl = [jnp.broadcast_to(att_vec[h], (16,)) for h in range(H)]
    ids0 = lax.iota(jnp.int32, 16)
    ids16 = ids0 * 16
    plsc.subcore_barrier()

    def _fire_loads(ci, buf):
        xlr, xrr, er, _, seml, _ = buf
        for j in range(NSUB):
            sl = pl.ds(j * SUB, SUB)
            pltpu.async_copy(
                xl_hbm.at[srcall.at[pl.ds(ci * K + j * SUB, SUB)]],
                xlr.at[sl], seml)
            pltpu.async_copy(
                xr_hbm.at[dstall.at[ci * NSUB + j]],
                xrr.at[sl], seml)
        pltpu.async_copy(
            e_hbm.at[pl.ds((base0 + ci * K) * H, K * H)], er, seml)

    def _wait_loads(buf):
        xlr, xrr, er, _, seml, _ = buf
        pltpu.make_async_copy(xl_hbm.at[pl.ds(0, K)], xlr, seml).wait()
        pltpu.make_async_copy(xr_hbm.at[pl.ds(0, K)], xrr, seml).wait()
        pltpu.make_async_copy(e_hbm.at[pl.ds(0, K * H)], er, seml).wait()

    def _fire_scatters(ci, buf):
        xlr, _, _, expb, _, sems = buf
        for j in range(NSUB):
            sl = pl.ds(j * SUB, SUB)
            pltpu.async_copy(expb.at[sl],
                             shared_den.at[dstall.at[ci * NSUB + j]],
                             sems, add=True)
            pltpu.async_copy(xlr.at[sl],
                             shared_out.at[dstall.at[ci * NSUB + j]],
                             sems, add=True)

    def _drain_scatters(buf):
        xlr, _, _, expb, _, sems = buf
        pltpu.make_async_copy(expb, shared_den.at[pl.ds(0, K)], sems).wait()
        pltpu.make_async_copy(xlr, shared_out.at[pl.ds(0, K)], sems).wait()

    def _compute(buf):
        xlr, xrr, er, expb, _, _ = buf

        def _grp(g, _):
            ids = ids0 + g * 16
            gbase = ids16 + g * 256
            accs = [_zero16() for _ in range(8)]
            vxl = []
            for h in range(H):
                hh = jnp.full((16,), h, jnp.int32)
                xv = plsc.load_gather(xlr, [ids, hh])
                vxl.append(xv)
                m = (xv
                     + plsc.load_gather(xrr, [ids, hh])
                     + plsc.load_gather(er, [gbase + h]))
                m = jnp.maximum(m, 0.2 * m)
                accs[h % 8] = accs[h % 8] + att_spl[h] * m
            w = jnp.exp(((accs[0] + accs[1]) + (accs[2] + accs[3]))
                        + ((accs[4] + accs[5]) + (accs[6] + accs[7])))
            expb[pl.ds(g * 16, 16)] = w
            for h in range(H):
                hh = jnp.full((16,), h, jnp.int32)
                plsc.store_scatter(xlr, [ids, hh], vxl[h] * w)
            return 0
        lax.fori_loop(0, K // 16, _grp, 0)

    _fire_loads(0, bufs[0])
    _fire_loads(1, bufs[1])

    def _chunk(ci, _):
        for r in range(_NBUF):
            @pl.when(lax.rem(ci, _NBUF) == r)
            def _():
                buf = bufs[r]
                _wait_loads(buf)
                _compute(buf)
                _fire_scatters(ci, buf)
                nxt = bufs[(r + 2) % _NBUF]

                @pl.when(ci + 2 < NCHUNK)
                def _():
                    @pl.when(ci >= 1)
                    def _():
                        _drain_scatters(nxt)
                    _fire_loads(ci + 2, nxt)
        return 0
    lax.fori_loop(0, NCHUNK, _chunk, 0)

    for r in range(_NBUF):
        _drain_scatters(bufs[r])

    plsc.subcore_barrier()

    @pl.when(s == 0)
    def _():
        pltpu.sync_copy(shared_den, denomp_hbm.at[c])
        pltpu.sync_copy(shared_out, outp_hbm.at[c])


def _bufset():
    return [
        pltpu.VMEM((K, H), jnp.float32),
        pltpu.VMEM((K, H), jnp.float32),
        pltpu.VMEM((K * H,), jnp.float32),
        pltpu.VMEM((K,), jnp.float32),
    ]


_sc_layer = pl.kernel(
    _sc_layer_body,
    out_type=(jax.ShapeDtypeStruct((NC, NPAD), jnp.float32),
              jax.ShapeDtypeStruct((NC, NPAD, H), jnp.float32)),
    mesh=_mesh,
    compiler_params=_sc_params,
    scratch_types=[
        pltpu.VMEM((EW,), jnp.int32),
        pltpu.VMEM((EW // SUB, SUB), jnp.int32),
        *_bufset(), *_bufset(), *_bufset(),
        pltpu.VMEM((H,), jnp.float32),
        pltpu.VMEM((ZR,), jnp.float32),
        pltpu.VMEM((ZR, H), jnp.float32),
        pltpu.VMEM_SHARED((NPAD,), jnp.float32),
        pltpu.VMEM_SHARED((NPAD, H), jnp.float32),
        pltpu.SemaphoreType.DMA,
        pltpu.SemaphoreType.DMA,
        pltpu.SemaphoreType.DMA,
        pltpu.SemaphoreType.DMA,
        pltpu.SemaphoreType.DMA,
        pltpu.SemaphoreType.DMA,
    ],
)


_EB8 = 5000
_RB = 2000


def _tc_node_body(x_ref, wl_ref, wr_ref, xl_ref, xr_ref):
    xb = x_ref[...]
    xl_ref[...] = jnp.dot(xb, wl_ref[...], preferred_element_type=jnp.float32)
    xr_ref[...] = jnp.dot(xb, wr_ref[...], preferred_element_type=jnp.float32)


_tc_node = pl.pallas_call(
    _tc_node_body,
    grid=(N // _RB,),
    in_specs=[
        pl.BlockSpec((_RB, DF), lambda i: (i, 0)),
        pl.BlockSpec((DF, H), lambda i: (0, 0)),
        pl.BlockSpec((DF, H), lambda i: (0, 0)),
    ],
    out_specs=[
        pl.BlockSpec((_RB, H), lambda i: (i, 0)),
        pl.BlockSpec((_RB, H), lambda i: (i, 0)),
    ],
    out_shape=[
        jax.ShapeDtypeStruct((N, H), jnp.float32),
        jax.ShapeDtypeStruct((N, H), jnp.float32),
    ],
)


def _tc_edge_body(ea_ref, wl_ref, wr_ref, e1_ref, e2_ref):
    ea = ea_ref[...].reshape(_EB8, DF)
    e1_ref[...] = jnp.dot(ea, wl_ref[...], preferred_element_type=jnp.float32)
    e2_ref[...] = jnp.dot(ea, wr_ref[...], preferred_element_type=jnp.float32)


_tc_edge = pl.pallas_call(
    _tc_edge_body,
    grid=(E // 8 // _EB8,),
    in_specs=[
        pl.BlockSpec((_EB8 * DF,), lambda i: (i,)),
        pl.BlockSpec((DF, DF), lambda i: (0, 0)),
        pl.BlockSpec((DF, DF), lambda i: (0, 0)),
    ],
    out_specs=[
        pl.BlockSpec((_EB8, DF), lambda i: (i, 0)),
        pl.BlockSpec((_EB8, DF), lambda i: (i, 0)),
    ],
    out_shape=[
        jax.ShapeDtypeStruct((E // 8, DF), jnp.float32),
        jax.ShapeDtypeStruct((E // 8, DF), jnp.float32),
    ],
)


def _tc_mid_body(p_ref, dp_ref, b1_ref, wl_ref, wr_ref, xl_ref, xr_ref):
    den = dp_ref[:, 0] + dp_ref[:, 1] + 1e-16
    h = (p_ref[0] + p_ref[1]) / den.reshape(-1, 1) + b1_ref[...]
    h = jnp.maximum(h, 0.0)
    xl_ref[...] = jnp.dot(h, wl_ref[...], preferred_element_type=jnp.float32)
    xr_ref[...] = jnp.dot(h, wr_ref[...], preferred_element_type=jnp.float32)


_tc_mid = pl.pallas_call(
    _tc_mid_body,
    grid=(N // _RB,),
    in_specs=[
        pl.BlockSpec((NC, _RB, H), lambda i: (0, i, 0)),
        pl.BlockSpec((_RB, NC), lambda i: (i, 0)),
        pl.BlockSpec((1, H), lambda i: (0, 0)),
        pl.BlockSpec((H, H), lambda i: (0, 0)),
        pl.BlockSpec((H, H), lambda i: (0, 0)),
    ],
    out_specs=[
        pl.BlockSpec((_RB, H), lambda i: (i, 0)),
        pl.BlockSpec((_RB, H), lambda i: (i, 0)),
    ],
    out_shape=[
        jax.ShapeDtypeStruct((N, H), jnp.float32),
        jax.ShapeDtypeStruct((N, H), jnp.float32),
    ],
)


def _tc_final_body(p_ref, dp_ref, b2_ref, out_ref):
    den = dp_ref[:, 0] + dp_ref[:, 1] + 1e-16
    zv = (p_ref[0] + p_ref[1]) / den.reshape(-1, 1) + b2_ref[...]
    col = lax.broadcasted_iota(jnp.int32, zv.shape, 1)
    valid = col < 6
    zm = jnp.where(valid, zv, -jnp.inf)
    mx = jnp.max(zm, axis=1, keepdims=True)
    ez = jnp.where(valid, jnp.exp(zv - mx), 0.0)
    lse = jnp.log(jnp.sum(ez, axis=1, keepdims=True))
    out_ref[...] = zv - mx - lse


_tc_final = pl.pallas_call(
    _tc_final_body,
    grid=(N // _RB,),
    in_specs=[
        pl.BlockSpec((NC, _RB, H), lambda i: (0, i, 0)),
        pl.BlockSpec((_RB, NC), lambda i: (i, 0)),
        pl.BlockSpec((1, H), lambda i: (0, 0)),
    ],
    out_specs=pl.BlockSpec((_RB, H), lambda i: (i, 0)),
    out_shape=jax.ShapeDtypeStruct((N, H), jnp.float32),
)


def _blockdiag8(w):
    return jnp.kron(jnp.eye(8, dtype=jnp.float32), w)


def kernel(x, edge_index, edge_attr, Wl1, Wr1, We1, att1, b1,
           Wl2, Wr2, We2, att2, b2):
    src = edge_index[0]
    dst = edge_index[1]
    dst2d = dst.reshape(E // SUB, SUB)

    xl1, xr1 = _tc_node(x, Wl1, Wr1)

    We2p = jnp.pad(We2, ((0, 0), (0, H - We2.shape[1])))
    e1r, e2r = _tc_edge(edge_attr.reshape(E * 16),
                        _blockdiag8(We1), _blockdiag8(We2p))

    dp1, outp1 = _sc_layer(xl1, xr1, e1r.reshape(E * H), src, dst2d, att1)

    Wl2p = jnp.pad(Wl2, ((0, 0), (0, H - Wl2.shape[1])))
    Wr2p = jnp.pad(Wr2, ((0, 0), (0, H - Wr2.shape[1])))
    xl2, xr2 = _tc_mid(outp1, dp1.T, b1.reshape(1, H), Wl2p, Wr2p)

    att2p = jnp.pad(att2, (0, H - att2.shape[0]))
    dp2, outp2 = _sc_layer(xl2, xr2, e2r.reshape(E * H), src, dst2d, att2p)

    b2p = jnp.pad(b2, (0, H - b2.shape[0])).reshape(1, H)
    out16 = _tc_final(outp2, dp2.T, b2p)
    return out16[:, :6]


# speedup vs baseline: 1.0073x; 1.0073x over previous
import jax
import jax.numpy as jnp
from jax import lax
from jax.experimental import pallas as pl
from jax.experimental.pallas import tpu as pltpu
from jax.experimental.pallas import tpu_sc as plsc

N = 10000
E = 320000
DF = 128
H = 16
NPAD = 10240
NC = 2
NS = 16
NW = NC * NS
EW = E // NW
K = 400
SUB = 80
NSUB = K // SUB
NCHUNK = EW // K
ZR = NPAD // NS

_mesh = plsc.VectorSubcoreMesh(
    core_axis_name="c", subcore_axis_name="s", num_cores=NC, num_subcores=NS)
_sc_params = pltpu.CompilerParams(
    needs_layout_passes=False, use_tc_tiling_on_sc=False)


def _zero16():
    return jnp.zeros((16,), jnp.float32)


_NBUF = 3


def _sc_layer_body(xl_hbm, xr_hbm, e_hbm, src_hbm, dst_hbm, att_hbm,
                   denomp_hbm, outp_hbm,
                   srcall, dstall,
                   xlr0, xrr0, er0, expb0,
                   xlr1, xrr1, er1, expb1,
                   xlr2, xrr2, er2, expb2,
                   attv, zbuf, zrowb, shared_den, shared_out,
                   seml0, seml1, seml2, sems0, sems1, sems2):
    c = lax.axis_index("c")
    s = lax.axis_index("s")
    wid = c * NS + s
    base0 = wid * EW

    bufs = ((xlr0, xrr0, er0, expb0, seml0, sems0),
            (xlr1, xrr1, er1, expb1, seml1, sems1),
            (xlr2, xrr2, er2, expb2, seml2, sems2))

    pltpu.sync_copy(src_hbm.at[pl.ds(base0, EW)], srcall)
    pltpu.sync_copy(dst_hbm.at[pl.ds(base0 // SUB, EW // SUB)], dstall)

    z = _zero16()

    def _zb(i, _):
        zbuf[pl.ds(i * 16, 16)] = z
        return 0
    lax.fori_loop(0, ZR // 16, _zb, 0)

    def _zr(i, _):
        zrowb[i, :] = z
        return 0
    lax.fori_loop(0, ZR, _zr, 0)
    pltpu.sync_copy(zbuf, shared_den.at[pl.ds(s * ZR, ZR)])
    pltpu.sync_copy(zrowb, shared_out.at[pl.ds(s * ZR, ZR)])

    pltpu.sync_copy(att_hbm, attv)
    att_vec = attv[...]
    att_spl = [jnp.broadcast_to(att_vec[h], (16,)) for h in range(H)]
    ids0 = lax.iota(jnp.int32, 16)
    ids16 = ids0 * 16
    plsc.subcore_barrier()

    def _fire_loads(ci, buf):
        xlr, xrr, er, _, seml, _ = buf
        for j in range(NSUB):
            sl = pl.ds(j * SUB, SUB)
            pltpu.async_copy(
                xl_hbm.at[srcall.at[pl.ds(ci * K + j * SUB, SUB)]],
                xlr.at[sl], seml)
            pltpu.async_copy(
                xr_hbm.at[dstall.at[ci * NSUB + j]],
                xrr.at[sl], seml)
        pltpu.async_copy(
            e_hbm.at[pl.ds((base0 + ci * K) * H, K * H)], er, seml)

    def _wait_loads(buf):
        xlr, xrr, er, _, seml, _ = buf
        pltpu.make_async_copy(xl_hbm.at[pl.ds(0, K)], xlr, seml).wait()
        pltpu.make_async_copy(xr_hbm.at[pl.ds(0, K)], xrr, seml).wait()
        pltpu.make_async_copy(e_hbm.at[pl.ds(0, K * H)], er, seml).wait()

    def _fire_scatters(ci, buf):
        xlr, _, _, expb, _, sems = buf
        for j in range(NSUB):
            sl = pl.ds(j * SUB, SUB)
            pltpu.async_copy(expb.at[sl],
                             shared_den.at[dstall.at[ci * NSUB + j]],
                             sems, add=True)
            pltpu.async_copy(xlr.at[sl],
                             shared_out.at[dstall.at[ci * NSUB + j]],
                             sems, add=True)

    def _drain_scatters(buf):
        xlr, _, _, expb, _, sems = buf
        pltpu.make_async_copy(expb, shared_den.at[pl.ds(0, K)], sems).wait()
        pltpu.make_async_copy(xlr, shared_out.at[pl.ds(0, K)], sems).wait()

    def _compute(buf):
        xlr, xrr, er, expb, _, _ = buf

        def _grp(g, _):
            ids = ids0 + g * 16
            gbase = ids16 + g * 256
            accs = [_zero16() for _ in range(4)]
            vxl = []
            for h in range(H):
                hh = jnp.full((16,), h, jnp.int32)
                xv = plsc.load_gather(xlr, [ids, hh])
                vxl.append(xv)
                m = (xv
                     + plsc.load_gather(xrr, [ids, hh])
                     + plsc.load_gather(er, [gbase + h]))
                m = jnp.maximum(m, 0.2 * m)
                accs[h % 4] = accs[h % 4] + att_spl[h] * m
            w = jnp.exp((accs[0] + accs[1]) + (accs[2] + accs[3]))
            expb[pl.ds(g * 16, 16)] = w
            for h in range(H):
                hh = jnp.full((16,), h, jnp.int32)
                plsc.store_scatter(xlr, [ids, hh], vxl[h] * w)
            return 0
        lax.fori_loop(0, K // 16, _grp, 0)

    _fire_loads(0, bufs[0])
    _fire_loads(1, bufs[1])

    def _chunk(ci, _):
        for r in range(_NBUF):
            @pl.when(lax.rem(ci, _NBUF) == r)
            def _():
                buf = bufs[r]
                _wait_loads(buf)
                _compute(buf)
                _fire_scatters(ci, buf)
                nxt = bufs[(r + 2) % _NBUF]

                @pl.when(ci + 2 < NCHUNK)
                def _():
                    @pl.when(ci >= 1)
                    def _():
                        _drain_scatters(nxt)
                    _fire_loads(ci + 2, nxt)
        return 0
    lax.fori_loop(0, NCHUNK, _chunk, 0)

    for r in range(_NBUF):
        _drain_scatters(bufs[r])

    plsc.subcore_barrier()

    @pl.when(s == 0)
    def _():
        pltpu.sync_copy(shared_den, denomp_hbm.at[c])
        pltpu.sync_copy(shared_out, outp_hbm.at[c])


def _bufset():
    return [
        pltpu.VMEM((K, H), jnp.float32),
        pltpu.VMEM((K, H), jnp.float32),
        pltpu.VMEM((K * H,), jnp.float32),
        pltpu.VMEM((K,), jnp.float32),
    ]


_sc_layer = pl.kernel(
    _sc_layer_body,
    out_type=(jax.ShapeDtypeStruct((NC, NPAD), jnp.float32),
              jax.ShapeDtypeStruct((NC, NPAD, H), jnp.float32)),
    mesh=_mesh,
    compiler_params=_sc_params,
    scratch_types=[
        pltpu.VMEM((EW,), jnp.int32),
        pltpu.VMEM((EW // SUB, SUB), jnp.int32),
        *_bufset(), *_bufset(), *_bufset(),
        pltpu.VMEM((H,), jnp.float32),
        pltpu.VMEM((ZR,), jnp.float32),
        pltpu.VMEM((ZR, H), jnp.float32),
        pltpu.VMEM_SHARED((NPAD,), jnp.float32),
        pltpu.VMEM_SHARED((NPAD, H), jnp.float32),
        pltpu.SemaphoreType.DMA,
        pltpu.SemaphoreType.DMA,
        pltpu.SemaphoreType.DMA,
        pltpu.SemaphoreType.DMA,
        pltpu.SemaphoreType.DMA,
        pltpu.SemaphoreType.DMA,
    ],
)


_EB8 = 5000
_RB = 2000


def _tc_node_body(x_ref, wl_ref, wr_ref, xl_ref, xr_ref):
    xb = x_ref[...]
    xl_ref[...] = jnp.dot(xb, wl_ref[...], preferred_element_type=jnp.float32)
    xr_ref[...] = jnp.dot(xb, wr_ref[...], preferred_element_type=jnp.float32)


_tc_node = pl.pallas_call(
    _tc_node_body,
    grid=(N // _RB,),
    in_specs=[
        pl.BlockSpec((_RB, DF), lambda i: (i, 0)),
        pl.BlockSpec((DF, H), lambda i: (0, 0)),
        pl.BlockSpec((DF, H), lambda i: (0, 0)),
    ],
    out_specs=[
        pl.BlockSpec((_RB, H), lambda i: (i, 0)),
        pl.BlockSpec((_RB, H), lambda i: (i, 0)),
    ],
    out_shape=[
        jax.ShapeDtypeStruct((N, H), jnp.float32),
        jax.ShapeDtypeStruct((N, H), jnp.float32),
    ],
)


def _tc_edge_body(ea_ref, wl_ref, wr_ref, e1_ref, e2_ref):
    ea = ea_ref[...].reshape(_EB8, DF)
    e1_ref[...] = jnp.dot(ea, wl_ref[...], preferred_element_type=jnp.float32)
    e2_ref[...] = jnp.dot(ea, wr_ref[...], preferred_element_type=jnp.float32)


_tc_edge = pl.pallas_call(
    _tc_edge_body,
    grid=(E // 8 // _EB8,),
    in_specs=[
        pl.BlockSpec((_EB8 * DF,), lambda i: (i,)),
        pl.BlockSpec((DF, DF), lambda i: (0, 0)),
        pl.BlockSpec((DF, DF), lambda i: (0, 0)),
    ],
    out_specs=[
        pl.BlockSpec((_EB8, DF), lambda i: (i, 0)),
        pl.BlockSpec((_EB8, DF), lambda i: (i, 0)),
    ],
    out_shape=[
        jax.ShapeDtypeStruct((E // 8, DF), jnp.float32),
        jax.ShapeDtypeStruct((E // 8, DF), jnp.float32),
    ],
)


def _tc_mid_body(p_ref, dp_ref, b1_ref, wl_ref, wr_ref, xl_ref, xr_ref):
    den = dp_ref[:, 0] + dp_ref[:, 1] + 1e-16
    h = (p_ref[0] + p_ref[1]) / den.reshape(-1, 1) + b1_ref[...]
    h = jnp.maximum(h, 0.0)
    xl_ref[...] = jnp.dot(h, wl_ref[...], preferred_element_type=jnp.float32)
    xr_ref[...] = jnp.dot(h, wr_ref[...], preferred_element_type=jnp.float32)


_tc_mid = pl.pallas_call(
    _tc_mid_body,
    grid=(N // _RB,),
    in_specs=[
        pl.BlockSpec((NC, _RB, H), lambda i: (0, i, 0)),
        pl.BlockSpec((_RB, NC), lambda i: (i, 0)),
        pl.BlockSpec((1, H), lambda i: (0, 0)),
        pl.BlockSpec((H, H), lambda i: (0, 0)),
        pl.BlockSpec((H, H), lambda i: (0, 0)),
    ],
    out_specs=[
        pl.BlockSpec((_RB, H), lambda i: (i, 0)),
        pl.BlockSpec((_RB, H), lambda i: (i, 0)),
    ],
    out_shape=[
        jax.ShapeDtypeStruct((N, H), jnp.float32),
        jax.ShapeDtypeStruct((N, H), jnp.float32),
    ],
)


def _tc_final_body(p_ref, dp_ref, b2_ref, out_ref):
    den = dp_ref[:, 0] + dp_ref[:, 1] + 1e-16
    zv = (p_ref[0] + p_ref[1]) / den.reshape(-1, 1) + b2_ref[...]
    col = lax.broadcasted_iota(jnp.int32, zv.shape, 1)
    valid = col < 6
    zm = jnp.where(valid, zv, -jnp.inf)
    mx = jnp.max(zm, axis=1, keepdims=True)
    ez = jnp.where(valid, jnp.exp(zv - mx), 0.0)
    lse = jnp.log(jnp.sum(ez, axis=1, keepdims=True))
    out_ref[...] = zv - mx - lse


_tc_final = pl.pallas_call(
    _tc_final_body,
    grid=(N // _RB,),
    in_specs=[
        pl.BlockSpec((NC, _RB, H), lambda i: (0, i, 0)),
        pl.BlockSpec((_RB, NC), lambda i: (i, 0)),
        pl.BlockSpec((1, H), lambda i: (0, 0)),
    ],
    out_specs=pl.BlockSpec((_RB, H), lambda i: (i, 0)),
    out_shape=jax.ShapeDtypeStruct((N, H), jnp.float32),
)


def _blockdiag8(w):
    return jnp.kron(jnp.eye(8, dtype=jnp.float32), w)


def kernel(x, edge_index, edge_attr, Wl1, Wr1, We1, att1, b1,
           Wl2, Wr2, We2, att2, b2):
    src = edge_index[0]
    dst = edge_index[1]
    dst2d = dst.reshape(E // SUB, SUB)

    xl1, xr1 = _tc_node(x, Wl1, Wr1)

    We2p = jnp.pad(We2, ((0, 0), (0, H - We2.shape[1])))
    e1r, e2r = _tc_edge(edge_attr.reshape(E * 16),
                        _blockdiag8(We1), _blockdiag8(We2p))

    dp1, outp1 = _sc_layer(xl1, xr1, e1r.reshape(E * H), src, dst2d, att1)

    Wl2p = jnp.pad(Wl2, ((0, 0), (0, H - Wl2.shape[1])))
    Wr2p = jnp.pad(Wr2, ((0, 0), (0, H - Wr2.shape[1])))
    xl2, xr2 = _tc_mid(outp1, dp1.T, b1.reshape(1, H), Wl2p, Wr2p)

    att2p = jnp.pad(att2, (0, H - att2.shape[0]))
    dp2, outp2 = _sc_layer(xl2, xr2, e2r.reshape(E * H), src, dst2d, att2p)

    b2p = jnp.pad(b2, (0, H - b2.shape[0])).reshape(1, H)
    out16 = _tc_final(outp2, dp2.T, b2p)
    return out16[:, :6]
